# baseline (speedup 1.0000x reference)
import functools

import jax
import jax.numpy as jnp
from jax import lax
from jax.experimental import pallas as pl
from jax.experimental.pallas import tpu as pltpu
from jax.experimental.pallas import tpu_sc as plsc

N = 100000
C = 256
G = 256
H = 16

NC = 2
NS = 16
NW = NC * NS

K = 64
NCHUNK = (N + K - 1) // K
TRASH = G
GP = 272
TPW = (NCHUNK + NW - 1) // NW

BLK = 2000
NBLK = N // BLK

_mesh = plsc.VectorSubcoreMesh(core_axis_name="c", subcore_axis_name="s")

GPV = 264


def _splat_lane(v, j):
    lane = lax.iota(jnp.int32, 16)
    s = jnp.sum(jnp.where(lane == j, v, 0), axis=0)
    return jnp.broadcast_to(s, (16,))


@functools.partial(
    pl.kernel,
    mesh=_mesh,
    compiler_params=pltpu.CompilerParams(needs_layout_passes=False),
    out_type=[
        jax.ShapeDtypeStruct((NW, G * C), jnp.float32),
        jax.ShapeDtypeStruct((NW, G * 16), jnp.float32),
    ],
    scratch_types=[
        pltpu.VMEM((K, C), jnp.float32),
        pltpu.VMEM((K,), jnp.int32),
        pltpu.VMEM((GPV * C,), jnp.float32),
        pltpu.VMEM((GPV * 16,), jnp.float32),
    ],
)
def _segsum_sc(x_hbm, bidx_hbm, zc_hbm, z16_hbm,
               acc_out, cnt_out, xv, iv, acc_f, cnt_f):
    cid = lax.axis_index("c")
    sid = lax.axis_index("s")
    wid = sid * NC + cid

    pltpu.sync_copy(zc_hbm, acc_f)
    pltpu.sync_copy(z16_hbm, cnt_f)
    ones16 = jnp.ones((16,), jnp.float32)
    lane = lax.iota(jnp.int32, 16)

    def chunk_body(t, _):
        chunk = wid * TPW + t

        @pl.when(chunk < NCHUNK)
        def _do():
            start = jnp.minimum(chunk * K, N - K)
            pltpu.sync_copy(bidx_hbm.at[chunk], iv)
            pltpu.sync_copy(x_hbm.at[pl.ds(start, K)], xv)

            for g in range(K // 16):
                b_vec = iv[pl.ds(g * 16, 16)]
                plsc.addupdate_scatter(cnt_f, [b_vec * 16 + lane], ones16)
                for j in range(16):
                    base = _splat_lane(b_vec, j) * C + lane
                    for l in range(C // 16):
                        plsc.addupdate_scatter(
                            acc_f, [base + l * 16],
                            xv[g * 16 + j, pl.ds(l * 16, 16)])
        return ()

    lax.fori_loop(0, TPW, chunk_body, ())

    pltpu.sync_copy(acc_f.at[pl.ds(0, G * C)], acc_out.at[wid])
    pltpu.sync_copy(cnt_f.at[pl.ds(0, G * 16)], cnt_out.at[wid])


def _mlp_body(acc_ref, cnt_ref, w1_ref, a_ref, w2_ref, s_ref):
    seg = jnp.sum(acc_ref[...], axis=0)
    cnt_full = jnp.sum(cnt_ref[...], axis=0)
    cnt = jnp.maximum(jnp.sum(cnt_full, axis=1, keepdims=True), 1.0)
    x_avg = seg / cnt
    h = lax.dot_general(x_avg, w1_ref[...], (((1,), (1,)), ((), ())),
                        preferred_element_type=jnp.float32)
    a = a_ref[0]
    h = jnp.where(h >= 0, h, a * h)
    h = lax.dot_general(h, w2_ref[...], (((1,), (1,)), ((), ())),
                        preferred_element_type=jnp.float32)
    s_ref[...] = jax.nn.sigmoid(h)


def _scale_body(x_ref, b_ref, s_ref, o_ref):
    b = b_ref[0, 0, :]
    gi = lax.broadcasted_iota(jnp.int32, (BLK, G), 1)
    oh = jnp.where(gi == b[:, None], 1.0, 0.0).astype(jnp.float32)
    se = lax.dot_general(oh, s_ref[...], (((1,), (0,)), ((), ())),
                         preferred_element_type=jnp.float32)
    o_ref[...] = x_ref[...] * se


def kernel(x, batch, num_graphs, W1, a, W2):
    bi = jnp.minimum(batch, num_graphs - 1).astype(jnp.int32)

    starts = jnp.minimum(jnp.arange(NCHUNK) * K, N - K)
    pos = starts[:, None] + jnp.arange(K)[None, :]
    vals = jnp.take(bi, pos.reshape(-1)).reshape(NCHUNK, K)
    fresh = pos >= (jnp.arange(NCHUNK) * K)[:, None]
    bidx = jnp.where(fresh, vals, TRASH).astype(jnp.int32)

    zc = jnp.zeros((GPV * C,), jnp.float32)
    z16 = jnp.zeros((GPV * 16,), jnp.float32)

    acc, cnt = _segsum_sc(x, bidx, zc, z16)
    acc = acc.reshape(NW, G, C)
    cnt = cnt.reshape(NW, G, 16)

    s = pl.pallas_call(
        _mlp_body,
        in_specs=[
            pl.BlockSpec((NW, G, C), lambda: (0, 0, 0)),
            pl.BlockSpec((NW, G, 16), lambda: (0, 0, 0)),
            pl.BlockSpec((H, C), lambda: (0, 0)),
            pl.BlockSpec(memory_space=pltpu.SMEM),
            pl.BlockSpec((C, H), lambda: (0, 0)),
        ],
        out_specs=pl.BlockSpec((G, C), lambda: (0, 0)),
        out_shape=jax.ShapeDtypeStruct((G, C), jnp.float32),
    )(acc, cnt, W1, a.reshape(1), W2)

    b3 = bi.reshape(NBLK, 1, BLK)
    out = pl.pallas_call(
        _scale_body,
        grid=(NBLK,),
        in_specs=[
            pl.BlockSpec((BLK, C), lambda i: (i, 0)),
            pl.BlockSpec((1, 1, BLK), lambda i: (i, 0, 0)),
            pl.BlockSpec((G, C), lambda i: (0, 0)),
        ],
        out_specs=pl.BlockSpec((BLK, C), lambda i: (i, 0)),
        out_shape=jax.ShapeDtypeStruct((N, C), jnp.float32),
    )(x, b3, s)
    return out
